# Initial kernel scaffold; baseline (speedup 1.0000x reference)
import jax
import jax.numpy as jnp
from jax.experimental import pallas as pl


def kernel(x, edge_index, batch, W1_0, b1_0, W2_0, b2_0, gamma_0, beta_0, W1_1, b1_1, W2_1, b2_1, gamma_1, beta_1):
    raise NotImplementedError("write your pallas kernel here")


# speedup vs baseline: 2.6918x; 2.6918x over previous
import jax
import jax.numpy as jnp
from jax import lax
from jax.experimental import pallas as pl
from jax.experimental.pallas import tpu as pltpu
from jax.experimental.pallas import tpu_sc as plsc

_N = 10000
_D = 128
_G = 16
_E = 320000
_EPS_BN = 1e-5

_NC = 2
_NS = 16
_NW = _NC * _NS
_EB = 128
_BW = 80
_TBP = _NW * _BW
_EPAD = _TBP * _EB - _E
_AR = 10240
_RPT = _AR // _NS
_ZR = 80


def _seg_sum_body(h_hbm, src_hbm, dst_hbm, out_hbm,
                  sidx, didx, rows, zbuf, acc, gsem):
    c = lax.axis_index("c")
    s = lax.axis_index("s")
    wid = s * _NC + c
    bbase = wid * _BW

    def zrow(r, carry):
        for k in range(_D // 16):
            zbuf[r, pl.ds(k * 16, 16)] = jnp.zeros((16,), jnp.float32)
        return carry
    lax.fori_loop(0, _ZR, zrow, 0)
    for t in range(_RPT // _ZR):
        pltpu.sync_copy(zbuf, acc.at[pl.ds(s * _RPT + t * _ZR, _ZR)])

    pltpu.sync_copy(src_hbm.at[pl.ds(bbase, _BW)], sidx)
    pltpu.sync_copy(dst_hbm.at[pl.ds(bbase, _BW)], didx)

    plsc.subcore_barrier()

    def body(j, carry):
        pltpu.async_copy(h_hbm.at[sidx.at[j]], rows, gsem).wait()
        pltpu.sync_copy(rows, acc.at[didx.at[j]], add=True)
        return carry
    lax.fori_loop(0, _BW, body, 0)

    plsc.subcore_barrier()

    @pl.when(s < _NS - 1)
    def _():
        pltpu.sync_copy(acc.at[pl.ds(s * _RPT, _RPT)],
                        out_hbm.at[c].at[pl.ds(s * _RPT, _RPT)])

    @pl.when(s == _NS - 1)
    def _():
        pltpu.sync_copy(acc.at[pl.ds((_NS - 1) * _RPT, _N - (_NS - 1) * _RPT)],
                        out_hbm.at[c].at[pl.ds((_NS - 1) * _RPT,
                                               _N - (_NS - 1) * _RPT)])


def _sc_segment_sum(h, src2, dst2):
    mesh = plsc.VectorSubcoreMesh(core_axis_name="c", subcore_axis_name="s")
    return pl.kernel(
        _seg_sum_body,
        out_type=jax.ShapeDtypeStruct((_NC, _N, _D), jnp.float32),
        mesh=mesh,
        scratch_types=[
            pltpu.VMEM((_BW, _EB), jnp.int32),
            pltpu.VMEM((_BW, _EB), jnp.int32),
            pltpu.VMEM((_EB, _D), jnp.float32),
            pltpu.VMEM((_ZR, _D), jnp.float32),
            pltpu.VMEM_SHARED((_AR, _D), jnp.float32),
            pltpu.SemaphoreType.DMA,
        ],
    )(h, src2, dst2)


def _tc_layer_body(x_ref, a_ref, w1_ref, b1_ref, w2_ref, b2_ref,
                   g_ref, be_ref, bt_ref, h_ref, p_ref):
    z = x_ref[...] + a_ref[0] + a_ref[1]
    u = jnp.dot(z.astype(jnp.bfloat16), w1_ref[...].astype(jnp.bfloat16),
                preferred_element_type=jnp.float32) + b1_ref[...]
    u = jnp.maximum(u, 0.0)
    u = jnp.dot(u.astype(jnp.bfloat16), w2_ref[...].astype(jnp.bfloat16),
                preferred_element_type=jnp.float32) + b2_ref[...]
    u = jnp.maximum(u, 0.0)
    mean = jnp.mean(u, axis=0, keepdims=True)
    d = u - mean
    var = jnp.mean(d * d, axis=0, keepdims=True)
    hn = g_ref[...] * d * lax.rsqrt(var + _EPS_BN) + be_ref[...]
    h_ref[...] = hn
    onehot = (bt_ref[...] == lax.broadcasted_iota(jnp.int32, (_G, _N), 0))
    p_ref[...] = jnp.dot(onehot.astype(jnp.float32), hn,
                         precision=lax.Precision.HIGHEST,
                         preferred_element_type=jnp.float32)


def _tc_layer(x, agg2, W1, b1, W2, b2, gamma, beta, bt):
    return pl.pallas_call(
        _tc_layer_body,
        out_shape=(jax.ShapeDtypeStruct((_N, _D), jnp.float32),
                   jax.ShapeDtypeStruct((_G, _D), jnp.float32)),
    )(x, agg2, W1, b1, W2, b2, gamma, beta, bt)


def kernel(x, edge_index, batch,
           W1_0, b1_0, W2_0, b2_0, gamma_0, beta_0,
           W1_1, b1_1, W2_1, b2_1, gamma_1, beta_1):
    src = edge_index[0].astype(jnp.int32)
    dst = edge_index[1].astype(jnp.int32)
    src2 = jnp.concatenate(
        [src, jnp.zeros((_EPAD,), jnp.int32)]).reshape(_TBP, _EB)
    dst2 = jnp.concatenate(
        [dst, jnp.full((_EPAD,), _N, jnp.int32)]).reshape(_TBP, _EB)
    bt = batch.astype(jnp.int32).reshape(1, _N)

    agg0 = _sc_segment_sum(x, src2, dst2)
    h0, p0 = _tc_layer(x, agg0, W1_0, b1_0.reshape(1, _D), W2_0,
                       b2_0.reshape(1, _D), gamma_0.reshape(1, _D),
                       beta_0.reshape(1, _D), bt)
    agg1 = _sc_segment_sum(h0, src2, dst2)
    h1, p1 = _tc_layer(h0, agg1, W1_1, b1_1.reshape(1, _D), W2_1,
                       b2_1.reshape(1, _D), gamma_1.reshape(1, _D),
                       beta_1.reshape(1, _D), bt)
    return (jnp.concatenate([p0, p1], axis=1), h1)


# speedup vs baseline: 2.9930x; 1.1119x over previous
import jax
import jax.numpy as jnp
from jax import lax
from jax.experimental import pallas as pl
from jax.experimental.pallas import tpu as pltpu
from jax.experimental.pallas import tpu_sc as plsc

_N = 10000
_D = 128
_G = 16
_E = 320000
_EPS_BN = 1e-5

_NC = 2
_NS = 16
_NW = _NC * _NS
_EB = 128
_BW = 80
_TBP = _NW * _BW
_EPAD = _TBP * _EB - _E
_AR = 10240
_RPT = _AR // _NS
_PB = _BW // 2


def _seg_sum_body(h_hbm, src_hbm, dst_hbm, out_hbm,
                  sidx, didx, rows, acc, gsem0, gsem1):
    c = lax.axis_index("c")
    s = lax.axis_index("s")
    wid = s * _NC + c
    bbase = wid * _BW

    def zrow(r, carry):
        for k in range(_D // 16):
            rows[0, r, pl.ds(k * 16, 16)] = jnp.zeros((16,), jnp.float32)
        return carry
    lax.fori_loop(0, _EB, zrow, 0)
    for t in range(_RPT // _EB):
        pltpu.sync_copy(rows.at[0], acc.at[pl.ds(s * _RPT + t * _EB, _EB)])

    plsc.subcore_barrier()

    sems = (gsem0, gsem1)
    for phase in range(2):
        pb = bbase + phase * _PB
        pltpu.sync_copy(src_hbm.at[pl.ds(pb, _PB)], sidx)
        pltpu.sync_copy(dst_hbm.at[pl.ds(pb, _PB)], didx)
        for b in range(2):
            pltpu.async_copy(h_hbm.at[sidx.at[b]], rows.at[b], sems[b])

        def body(t, carry):
            j0 = t * 2
            for b in range(2):
                j = j0 + b
                pltpu.make_async_copy(h_hbm.at[sidx.at[j]], rows.at[b],
                                      sems[b]).wait()
                pltpu.sync_copy(rows.at[b], acc.at[didx.at[j]], add=True)

                @pl.when(j + 2 < _PB)
                def _():
                    pltpu.async_copy(h_hbm.at[sidx.at[j + 2]], rows.at[b],
                                     sems[b])
            return carry
        lax.fori_loop(0, _PB // 2, body, 0)

    plsc.subcore_barrier()

    @pl.when(s < _NS - 1)
    def _():
        pltpu.sync_copy(acc.at[pl.ds(s * _RPT, _RPT)],
                        out_hbm.at[c].at[pl.ds(s * _RPT, _RPT)])

    @pl.when(s == _NS - 1)
    def _():
        pltpu.sync_copy(acc.at[pl.ds((_NS - 1) * _RPT, _N - (_NS - 1) * _RPT)],
                        out_hbm.at[c].at[pl.ds((_NS - 1) * _RPT,
                                               _N - (_NS - 1) * _RPT)])


def _sc_segment_sum(h, src2, dst2):
    mesh = plsc.VectorSubcoreMesh(core_axis_name="c", subcore_axis_name="s")
    return pl.kernel(
        _seg_sum_body,
        out_type=jax.ShapeDtypeStruct((_NC, _N, _D), jnp.float32),
        mesh=mesh,
        scratch_types=[
            pltpu.VMEM((_PB, _EB), jnp.int32),
            pltpu.VMEM((_PB, _EB), jnp.int32),
            pltpu.VMEM((2, _EB, _D), jnp.float32),
            pltpu.VMEM_SHARED((_AR, _D), jnp.float32),
            pltpu.SemaphoreType.DMA,
            pltpu.SemaphoreType.DMA,
        ],
    )(h, src2, dst2)


def _tc_layer_body(x_ref, a_ref, w1_ref, b1_ref, w2_ref, b2_ref,
                   g_ref, be_ref, bt_ref, h_ref, p_ref):
    z = x_ref[...] + a_ref[0] + a_ref[1]
    u = jnp.dot(z.astype(jnp.bfloat16), w1_ref[...].astype(jnp.bfloat16),
                preferred_element_type=jnp.float32) + b1_ref[...]
    u = jnp.maximum(u, 0.0)
    u = jnp.dot(u.astype(jnp.bfloat16), w2_ref[...].astype(jnp.bfloat16),
                preferred_element_type=jnp.float32) + b2_ref[...]
    u = jnp.maximum(u, 0.0)
    mean = jnp.mean(u, axis=0, keepdims=True)
    d = u - mean
    var = jnp.mean(d * d, axis=0, keepdims=True)
    hn = g_ref[...] * d * lax.rsqrt(var + _EPS_BN) + be_ref[...]
    h_ref[...] = hn
    onehot = (bt_ref[...] == lax.broadcasted_iota(jnp.int32, (_G, _N), 0))
    p_ref[...] = jnp.dot(onehot.astype(jnp.float32), hn,
                         precision=lax.Precision.HIGHEST,
                         preferred_element_type=jnp.float32)


def _tc_layer(x, agg2, W1, b1, W2, b2, gamma, beta, bt):
    return pl.pallas_call(
        _tc_layer_body,
        out_shape=(jax.ShapeDtypeStruct((_N, _D), jnp.float32),
                   jax.ShapeDtypeStruct((_G, _D), jnp.float32)),
    )(x, agg2, W1, b1, W2, b2, gamma, beta, bt)


def kernel(x, edge_index, batch,
           W1_0, b1_0, W2_0, b2_0, gamma_0, beta_0,
           W1_1, b1_1, W2_1, b2_1, gamma_1, beta_1):
    src = edge_index[0].astype(jnp.int32)
    dst = edge_index[1].astype(jnp.int32)
    src2 = jnp.concatenate(
        [src, jnp.zeros((_EPAD,), jnp.int32)]).reshape(_TBP, _EB)
    dst2 = jnp.concatenate(
        [dst, jnp.full((_EPAD,), _N, jnp.int32)]).reshape(_TBP, _EB)
    bt = batch.astype(jnp.int32).reshape(1, _N)

    agg0 = _sc_segment_sum(x, src2, dst2)
    h0, p0 = _tc_layer(x, agg0, W1_0, b1_0.reshape(1, _D), W2_0,
                       b2_0.reshape(1, _D), gamma_0.reshape(1, _D),
                       beta_0.reshape(1, _D), bt)
    agg1 = _sc_segment_sum(h0, src2, dst2)
    h1, p1 = _tc_layer(h0, agg1, W1_1, b1_1.reshape(1, _D), W2_1,
                       b2_1.reshape(1, _D), gamma_1.reshape(1, _D),
                       beta_1.reshape(1, _D), bt)
    return (jnp.concatenate([p0, p1], axis=1), h1)
